# baseline (speedup 1.0000x reference)
import functools

import jax
import jax.numpy as jnp
from jax import lax
from jax.experimental import pallas as pl
from jax.experimental.pallas import tpu as pltpu
from jax.experimental.pallas import tpu_sc as plsc

NC, NS, LANES = 2, 16, 16
N_NODES = 50000
N_PAD = 50176
ROWS_PER_TILE = N_PAD // NS
R = 5
E = 640000
D = 8
CHUNK = 128
EP = 655360
PER_TILE = EP // NS
N_CHUNKS = PER_TILE // CHUNK
HALF_CHUNKS = N_CHUNKS // 2
KBUF = 8
KD = 16
STRIP = 392
SPT = ROWS_PER_TILE // STRIP


def _sc_degrees(esrc4, edst4, ones_hbm, zvec_hbm):
    mesh = plsc.VectorSubcoreMesh(core_axis_name="c", subcore_axis_name="s")

    @functools.partial(
        pl.kernel,
        mesh=mesh,
        out_type=[jax.ShapeDtypeStruct((N_PAD,), jnp.float32),
                  jax.ShapeDtypeStruct((N_PAD,), jnp.float32)],
        scratch_types=[
            pltpu.VMEM((N_CHUNKS, CHUNK), jnp.int32),
            pltpu.VMEM((CHUNK,), jnp.float32),
            pltpu.VMEM((ROWS_PER_TILE,), jnp.float32),
            pltpu.VMEM_SHARED((N_PAD,), jnp.float32),
            pltpu.SemaphoreType.DMA,
        ],
    )
    def deg_kernel(src_hbm, dst_hbm, ones_hbm_ref, zvec_ref, degu_hbm, degi_hbm,
                   idx2d, ones_v, stage_v, acc_sh, sem):
        cid = lax.axis_index("c")
        sid = lax.axis_index("s")
        row0 = sid * ROWS_PER_TILE

        pltpu.sync_copy(ones_hbm_ref, ones_v)
        pltpu.sync_copy(zvec_ref, stage_v)
        pltpu.sync_copy(stage_v, acc_sh.at[pl.ds(row0, ROWS_PER_TILE)])
        plsc.subcore_barrier()

        def run(e_hbm, out_hbm):
            for r in range(R):
                pltpu.sync_copy(e_hbm.at[r, sid], idx2d)

                @pl.loop(0, N_CHUNKS, step=KD)
                def _(j0):
                    for b in range(KD):
                        pltpu.async_copy(ones_v, acc_sh.at[idx2d.at[j0 + b]],
                                         sem, add=True)
                    for b in range(KD):
                        pltpu.make_async_copy(
                            ones_v, acc_sh.at[idx2d.at[j0 + b]], sem).wait()

            plsc.subcore_barrier()
            pltpu.sync_copy(acc_sh.at[pl.ds(row0, ROWS_PER_TILE)], stage_v)
            pltpu.sync_copy(stage_v, out_hbm.at[pl.ds(row0, ROWS_PER_TILE)])

        @pl.when(cid == 0)
        def _():
            run(src_hbm, degu_hbm)

        @pl.when(cid == 1)
        def _():
            run(dst_hbm, degi_hbm)

    return deg_kernel(esrc4, edst4, ones_hbm, zvec_hbm)


def _sc_edges(hu_tab, hi_tab, gsrc4, gdst4, esrc4, edst4, zrows_hbm):
    mesh = plsc.VectorSubcoreMesh(core_axis_name="c", subcore_axis_name="s")

    @functools.partial(
        pl.kernel,
        mesh=mesh,
        out_type=[jax.ShapeDtypeStruct((N_PAD, 128), jnp.float32),
                  jax.ShapeDtypeStruct((N_PAD, 128), jnp.float32)],
        scratch_types=[
            pltpu.VMEM((HALF_CHUNKS, CHUNK), jnp.int32),
            pltpu.VMEM((HALF_CHUNKS, CHUNK), jnp.int32),
            [pltpu.VMEM((CHUNK, D), jnp.float32) for _ in range(2 * KBUF)],
            pltpu.VMEM((STRIP, D), jnp.float32),
            pltpu.VMEM_SHARED((N_PAD, D), jnp.float32),
            pltpu.SemaphoreType.DMA,
            pltpu.SemaphoreType.DMA,
        ],
        compiler_params=pltpu.CompilerParams(use_tc_tiling_on_sc=False),
    )
    def edge_kernel(hu_hbm, hi_hbm, gsrc_hbm, gdst_hbm, esrc_hbm, edst_hbm,
                    zrows_ref, aggu_hbm, aggi_hbm, gidx2d, sidx2d, rows,
                    stage_v, acc_sh, gsem, ssem):
        cid = lax.axis_index("c")
        sid = lax.axis_index("s")
        row0 = sid * ROWS_PER_TILE
        half_a = rows[:KBUF]
        half_b = rows[KBUF:]

        def issue_gathers(h_hbm, j0, half):
            for b in range(KBUF):
                pltpu.async_copy(h_hbm.at[gidx2d.at[j0 + b]], half[b], gsem)

        def wait_scatters(half):
            for b in range(KBUF):
                pltpu.make_async_copy(
                    half[b], acc_sh.at[sidx2d.at[0]], ssem).wait()

        def drain_issue(h_hbm, j0, half):
            for b in range(KBUF):
                pltpu.make_async_copy(h_hbm.at[gidx2d.at[j0 + b]],
                                      half[b], gsem).wait()
                pltpu.async_copy(half[b], acc_sh.at[sidx2d.at[j0 + b]],
                                 ssem, add=True)

        def run(h_hbm, g_hbm, s_hbm, out_hbm):
            pltpu.sync_copy(zrows_ref, stage_v)
            for k in range(SPT):
                pltpu.sync_copy(
                    stage_v, acc_sh.at[pl.ds(row0 + k * STRIP, STRIP)])
            plsc.subcore_barrier()

            @pl.loop(0, 2 * R)
            def _(rp):
                r = rp // 2
                p = rp % 2
                pltpu.sync_copy(
                    g_hbm.at[r, sid, pl.ds(p * HALF_CHUNKS, HALF_CHUNKS)],
                    gidx2d)
                pltpu.sync_copy(
                    s_hbm.at[r, sid, pl.ds(p * HALF_CHUNKS, HALF_CHUNKS)],
                    sidx2d)

                issue_gathers(h_hbm, 0, half_a)

                @pl.loop(0, HALF_CHUNKS // (2 * KBUF))
                def _(t):
                    ja = t * (2 * KBUF)
                    jb = ja + KBUF

                    @pl.when(ja > 0)
                    def _():
                        wait_scatters(half_b)

                    issue_gathers(h_hbm, jb, half_b)
                    drain_issue(h_hbm, ja, half_a)

                    wait_scatters(half_a)

                    @pl.when(jb + KBUF < HALF_CHUNKS)
                    def _():
                        issue_gathers(h_hbm, jb + KBUF, half_a)

                    drain_issue(h_hbm, jb, half_b)

                wait_scatters(half_b)

                @pl.when(p == 1)
                def _():
                    plsc.subcore_barrier()
                    for k in range(SPT):
                        pltpu.async_copy(
                            acc_sh.at[pl.ds(row0 + k * STRIP, STRIP)],
                            out_hbm.at[pl.ds(row0 + k * STRIP, STRIP),
                                       pl.ds(r * D, D)], gsem)
                    for k in range(SPT):
                        pltpu.make_async_copy(
                            acc_sh.at[pl.ds(row0 + k * STRIP, STRIP)],
                            out_hbm.at[pl.ds(row0 + k * STRIP, STRIP),
                                       pl.ds(r * D, D)], gsem).wait()

                    @pl.when(rp < 2 * R - 1)
                    def _():
                        for k in range(SPT):
                            pltpu.async_copy(
                                stage_v,
                                acc_sh.at[pl.ds(row0 + k * STRIP, STRIP)],
                                ssem)
                        for k in range(SPT):
                            pltpu.make_async_copy(
                                stage_v,
                                acc_sh.at[pl.ds(row0 + k * STRIP, STRIP)],
                                ssem).wait()

                    plsc.subcore_barrier()

        @pl.when(cid == 0)
        def _():
            run(hu_hbm, gsrc_hbm, edst_hbm, aggi_hbm)

        @pl.when(cid == 1)
        def _():
            run(hi_hbm, gdst_hbm, esrc_hbm, aggu_hbm)

    return edge_kernel(hu_tab, hi_tab, gsrc4, gdst4, esrc4, edst4, zrows_hbm)


def _tc_prep(feat_pad, Wcat, deg):
    blk = 6272

    def body(x_ref, w_ref, d_ref, o_ref):
        c = lax.rsqrt(jnp.maximum(d_ref[...], 1.0))
        h = jnp.dot(x_ref[...], w_ref[...],
                    preferred_element_type=jnp.float32) * c
        o_ref[...] = jnp.concatenate(
            [h, jnp.zeros((blk, 128 - R * D), jnp.float32)], axis=1)

    return pl.pallas_call(
        body,
        grid=(N_PAD // blk,),
        in_specs=[
            pl.BlockSpec((blk, 128), lambda i: (i, 0)),
            pl.BlockSpec((128, R * D), lambda i: (0, 0)),
            pl.BlockSpec((blk, 1), lambda i: (i, 0)),
        ],
        out_specs=pl.BlockSpec((blk, 128), lambda i: (i, 0)),
        out_shape=jax.ShapeDtypeStruct((N_PAD, 128), jnp.float32),
    )(feat_pad, Wcat, deg)


def _tc_finish(agg128, deg, Wfc, bfc):
    blk = 6272

    def body(a_ref, d_ref, w_ref, b_ref, o_ref):
        c = lax.rsqrt(jnp.maximum(d_ref[...], 1.0))
        x = a_ref[:, :R * D]
        o_ref[...] = jnp.dot(x, w_ref[...],
                             preferred_element_type=jnp.float32) * c + b_ref[...]

    return pl.pallas_call(
        body,
        grid=(N_PAD // blk,),
        in_specs=[
            pl.BlockSpec((blk, 128), lambda i: (i, 0)),
            pl.BlockSpec((blk, 1), lambda i: (i, 0)),
            pl.BlockSpec((R * D, 64), lambda i: (0, 0)),
            pl.BlockSpec((1, 64), lambda i: (0, 0)),
        ],
        out_specs=pl.BlockSpec((blk, 64), lambda i: (i, 0)),
        out_shape=jax.ShapeDtypeStruct((N_NODES, 64), jnp.float32),
    )(agg128, deg, Wfc, bfc)


def kernel(ufeat, ifeat, edge_src, edge_dst, W_user, W_item,
           ufc_W, ufc_b, ifc_W, ifc_b):
    ufeat_p = jnp.pad(ufeat, ((0, N_PAD - N_NODES), (0, 0)))
    ifeat_p = jnp.pad(ifeat, ((0, N_PAD - N_NODES), (0, 0)))
    pad = jnp.full((R, EP - E), N_NODES, jnp.int32)
    esrc_p = jnp.concatenate([edge_src, pad], axis=1)
    edst_p = jnp.concatenate([edge_dst, pad], axis=1)
    esrc4 = esrc_p.reshape(R, NS, N_CHUNKS, CHUNK)
    edst4 = edst_p.reshape(R, NS, N_CHUNKS, CHUNK)
    ridx = jnp.arange(R, dtype=jnp.int32).reshape(R, 1)
    gsrc4 = (esrc_p * 16 + ridx).reshape(R, NS, N_CHUNKS, CHUNK)
    gdst4 = (edst_p * 16 + ridx).reshape(R, NS, N_CHUNKS, CHUNK)
    ones_hbm = jnp.ones((CHUNK,), jnp.float32)
    zvec_hbm = jnp.zeros((ROWS_PER_TILE,), jnp.float32)
    zrows_hbm = jnp.zeros((STRIP, D), jnp.float32)

    deg_u, deg_i = _sc_degrees(esrc4, edst4, ones_hbm, zvec_hbm)
    deg_u2 = deg_u.reshape(N_PAD, 1)
    deg_i2 = deg_i.reshape(N_PAD, 1)

    Wu_cat = jnp.transpose(W_user, (1, 0, 2)).reshape(128, R * D)
    Wi_cat = jnp.transpose(W_item, (1, 0, 2)).reshape(128, R * D)
    hu_tab = _tc_prep(ufeat_p, Wu_cat, deg_u2).reshape(N_PAD * 16, D)
    hi_tab = _tc_prep(ifeat_p, Wi_cat, deg_i2).reshape(N_PAD * 16, D)

    agg_u, agg_i = _sc_edges(hu_tab, hi_tab, gsrc4, gdst4, esrc4, edst4,
                             zrows_hbm)

    new_u = _tc_finish(agg_u, deg_u2, ufc_W, ufc_b.reshape(1, 64))
    new_i = _tc_finish(agg_i, deg_i2, ifc_W, ifc_b.reshape(1, 64))
    return new_u, new_i


# speedup vs baseline: 1.0531x; 1.0531x over previous
import functools

import jax
import jax.numpy as jnp
from jax import lax
from jax.experimental import pallas as pl
from jax.experimental.pallas import tpu as pltpu
from jax.experimental.pallas import tpu_sc as plsc

NC, NS, LANES = 2, 16, 16
N_NODES = 50000
N_PAD = 50176
ROWS_PER_TILE = N_PAD // NS
R = 5
E = 640000
D = 8
CHUNK = 128
EP = 655360
PER_TILE = EP // NS
N_CHUNKS = PER_TILE // CHUNK
HALF_CHUNKS = N_CHUNKS // 2
KBUF = 8
KD = 16
STRIP = 392
SPT = ROWS_PER_TILE // STRIP


def _sc_degrees(esrc4, edst4, ones_hbm, zvec_hbm):
    mesh = plsc.VectorSubcoreMesh(core_axis_name="c", subcore_axis_name="s")

    @functools.partial(
        pl.kernel,
        mesh=mesh,
        out_type=[jax.ShapeDtypeStruct((N_PAD,), jnp.float32),
                  jax.ShapeDtypeStruct((N_PAD,), jnp.float32)],
        scratch_types=[
            pltpu.VMEM((N_CHUNKS, CHUNK), jnp.int32),
            pltpu.VMEM((CHUNK,), jnp.float32),
            pltpu.VMEM((ROWS_PER_TILE,), jnp.float32),
            pltpu.VMEM_SHARED((N_PAD,), jnp.float32),
            pltpu.SemaphoreType.DMA,
        ],
    )
    def deg_kernel(src_hbm, dst_hbm, ones_hbm_ref, zvec_ref, degu_hbm, degi_hbm,
                   idx2d, ones_v, stage_v, acc_sh, sem):
        cid = lax.axis_index("c")
        sid = lax.axis_index("s")
        row0 = sid * ROWS_PER_TILE

        pltpu.sync_copy(ones_hbm_ref, ones_v)
        pltpu.sync_copy(zvec_ref, stage_v)
        pltpu.sync_copy(stage_v, acc_sh.at[pl.ds(row0, ROWS_PER_TILE)])
        plsc.subcore_barrier()

        def run(e_hbm, out_hbm):
            for r in range(R):
                pltpu.sync_copy(e_hbm.at[r, sid], idx2d)

                @pl.loop(0, N_CHUNKS, step=KD)
                def _(j0):
                    for b in range(KD):
                        pltpu.async_copy(ones_v, acc_sh.at[idx2d.at[j0 + b]],
                                         sem, add=True)
                    for b in range(KD):
                        pltpu.make_async_copy(
                            ones_v, acc_sh.at[idx2d.at[j0 + b]], sem).wait()

            plsc.subcore_barrier()
            pltpu.sync_copy(acc_sh.at[pl.ds(row0, ROWS_PER_TILE)], stage_v)
            pltpu.sync_copy(stage_v, out_hbm.at[pl.ds(row0, ROWS_PER_TILE)])

        @pl.when(cid == 0)
        def _():
            run(src_hbm, degu_hbm)

        @pl.when(cid == 1)
        def _():
            run(dst_hbm, degi_hbm)

    return deg_kernel(esrc4, edst4, ones_hbm, zvec_hbm)


def _sc_edges(hu_tab, hi_tab, gsrc4, gdst4, esrc4, edst4, zrows_hbm):
    mesh = plsc.VectorSubcoreMesh(core_axis_name="c", subcore_axis_name="s")

    @functools.partial(
        pl.kernel,
        mesh=mesh,
        out_type=[jax.ShapeDtypeStruct((N_PAD, 128), jnp.float32),
                  jax.ShapeDtypeStruct((N_PAD, 128), jnp.float32)],
        scratch_types=[
            pltpu.VMEM((N_CHUNKS, CHUNK), jnp.int32),
            pltpu.VMEM((N_CHUNKS, CHUNK), jnp.int32),
            [pltpu.VMEM((CHUNK, D), jnp.float32) for _ in range(2 * KBUF)],
            pltpu.VMEM((STRIP, D), jnp.float32),
            pltpu.VMEM_SHARED((N_PAD, D), jnp.float32),
            pltpu.SemaphoreType.DMA,
            pltpu.SemaphoreType.DMA,
        ],
        compiler_params=pltpu.CompilerParams(use_tc_tiling_on_sc=False),
    )
    def edge_kernel(hu_hbm, hi_hbm, gsrc_hbm, gdst_hbm, esrc_hbm, edst_hbm,
                    zrows_ref, aggu_hbm, aggi_hbm, gidx2d, sidx2d, rows,
                    stage_v, acc_sh, gsem, ssem):
        cid = lax.axis_index("c")
        sid = lax.axis_index("s")
        row0 = sid * ROWS_PER_TILE
        half_a = rows[:KBUF]
        half_b = rows[KBUF:]

        def issue_gathers(h_hbm, j0, half):
            for b in range(KBUF):
                pltpu.async_copy(h_hbm.at[gidx2d.at[j0 + b]], half[b], gsem)

        def wait_scatters(half):
            for b in range(KBUF):
                pltpu.make_async_copy(
                    half[b], acc_sh.at[sidx2d.at[0]], ssem).wait()

        def drain_issue(h_hbm, j0, half):
            for b in range(KBUF):
                pltpu.make_async_copy(h_hbm.at[gidx2d.at[j0 + b]],
                                      half[b], gsem).wait()
                pltpu.async_copy(half[b], acc_sh.at[sidx2d.at[j0 + b]],
                                 ssem, add=True)

        def run(h_hbm, g_hbm, s_hbm, out_hbm):
            for r in range(R):
                pltpu.sync_copy(zrows_ref, stage_v)
                for k in range(SPT):
                    pltpu.sync_copy(
                        stage_v, acc_sh.at[pl.ds(row0 + k * STRIP, STRIP)])
                plsc.subcore_barrier()
                pltpu.sync_copy(g_hbm.at[r, sid], gidx2d)
                pltpu.sync_copy(s_hbm.at[r, sid], sidx2d)

                issue_gathers(h_hbm, 0, half_a)

                @pl.loop(0, N_CHUNKS // (2 * KBUF))
                def _(t):
                    ja = t * (2 * KBUF)
                    jb = ja + KBUF

                    @pl.when(ja > 0)
                    def _():
                        wait_scatters(half_b)

                    issue_gathers(h_hbm, jb, half_b)
                    drain_issue(h_hbm, ja, half_a)

                    wait_scatters(half_a)

                    @pl.when(jb + KBUF < N_CHUNKS)
                    def _():
                        issue_gathers(h_hbm, jb + KBUF, half_a)

                    drain_issue(h_hbm, jb, half_b)

                wait_scatters(half_b)
                plsc.subcore_barrier()
                for k in range(SPT):
                    pltpu.sync_copy(
                        acc_sh.at[pl.ds(row0 + k * STRIP, STRIP)], stage_v)
                    pltpu.sync_copy(
                        stage_v,
                        out_hbm.at[pl.ds(row0 + k * STRIP, STRIP),
                                   pl.ds(r * D, D)])

        @pl.when(cid == 0)
        def _():
            run(hu_hbm, gsrc_hbm, edst_hbm, aggi_hbm)

        @pl.when(cid == 1)
        def _():
            run(hi_hbm, gdst_hbm, esrc_hbm, aggu_hbm)

    return edge_kernel(hu_tab, hi_tab, gsrc4, gdst4, esrc4, edst4, zrows_hbm)


def _tc_prep(feat_pad, Wcat, deg):
    blk = 6272

    def body(x_ref, w_ref, d_ref, o_ref):
        c = lax.rsqrt(jnp.maximum(d_ref[...], 1.0))
        h = jnp.dot(x_ref[...], w_ref[...],
                    preferred_element_type=jnp.float32) * c
        o_ref[...] = jnp.concatenate(
            [h, jnp.zeros((blk, 128 - R * D), jnp.float32)], axis=1)

    return pl.pallas_call(
        body,
        grid=(N_PAD // blk,),
        in_specs=[
            pl.BlockSpec((blk, 128), lambda i: (i, 0)),
            pl.BlockSpec((128, R * D), lambda i: (0, 0)),
            pl.BlockSpec((blk, 1), lambda i: (i, 0)),
        ],
        out_specs=pl.BlockSpec((blk, 128), lambda i: (i, 0)),
        out_shape=jax.ShapeDtypeStruct((N_PAD, 128), jnp.float32),
    )(feat_pad, Wcat, deg)


def _tc_finish(agg128, deg, Wfc, bfc):
    blk = 6272

    def body(a_ref, d_ref, w_ref, b_ref, o_ref):
        c = lax.rsqrt(jnp.maximum(d_ref[...], 1.0))
        x = a_ref[:, :R * D]
        o_ref[...] = jnp.dot(x, w_ref[...],
                             preferred_element_type=jnp.float32) * c + b_ref[...]

    return pl.pallas_call(
        body,
        grid=(N_PAD // blk,),
        in_specs=[
            pl.BlockSpec((blk, 128), lambda i: (i, 0)),
            pl.BlockSpec((blk, 1), lambda i: (i, 0)),
            pl.BlockSpec((R * D, 64), lambda i: (0, 0)),
            pl.BlockSpec((1, 64), lambda i: (0, 0)),
        ],
        out_specs=pl.BlockSpec((blk, 64), lambda i: (i, 0)),
        out_shape=jax.ShapeDtypeStruct((N_NODES, 64), jnp.float32),
    )(agg128, deg, Wfc, bfc)


def kernel(ufeat, ifeat, edge_src, edge_dst, W_user, W_item,
           ufc_W, ufc_b, ifc_W, ifc_b):
    ufeat_p = jnp.pad(ufeat, ((0, N_PAD - N_NODES), (0, 0)))
    ifeat_p = jnp.pad(ifeat, ((0, N_PAD - N_NODES), (0, 0)))
    pad = jnp.full((R, EP - E), N_NODES, jnp.int32)
    esrc_p = jnp.concatenate([edge_src, pad], axis=1)
    edst_p = jnp.concatenate([edge_dst, pad], axis=1)
    esrc4 = esrc_p.reshape(R, NS, N_CHUNKS, CHUNK)
    edst4 = edst_p.reshape(R, NS, N_CHUNKS, CHUNK)
    ridx = jnp.arange(R, dtype=jnp.int32).reshape(R, 1)
    gsrc4 = (esrc_p * 16 + ridx).reshape(R, NS, N_CHUNKS, CHUNK)
    gdst4 = (edst_p * 16 + ridx).reshape(R, NS, N_CHUNKS, CHUNK)
    ones_hbm = jnp.ones((CHUNK,), jnp.float32)
    zvec_hbm = jnp.zeros((ROWS_PER_TILE,), jnp.float32)
    zrows_hbm = jnp.zeros((STRIP, D), jnp.float32)

    deg_u, deg_i = _sc_degrees(esrc4, edst4, ones_hbm, zvec_hbm)
    deg_u2 = deg_u.reshape(N_PAD, 1)
    deg_i2 = deg_i.reshape(N_PAD, 1)

    Wu_cat = jnp.transpose(W_user, (1, 0, 2)).reshape(128, R * D)
    Wi_cat = jnp.transpose(W_item, (1, 0, 2)).reshape(128, R * D)
    hu_tab = _tc_prep(ufeat_p, Wu_cat, deg_u2).reshape(N_PAD * 16, D)
    hi_tab = _tc_prep(ifeat_p, Wi_cat, deg_i2).reshape(N_PAD * 16, D)

    agg_u, agg_i = _sc_edges(hu_tab, hi_tab, gsrc4, gdst4, esrc4, edst4,
                             zrows_hbm)

    new_u = _tc_finish(agg_u, deg_u2, ufc_W, ufc_b.reshape(1, 64))
    new_i = _tc_finish(agg_i, deg_i2, ifc_W, ifc_b.reshape(1, 64))
    return new_u, new_i
